# Initial kernel scaffold; baseline (speedup 1.0000x reference)
import jax
import jax.numpy as jnp
from jax.experimental import pallas as pl


def kernel(xyz, points, W0, b0, g0, be0, W1, b1, g1, be1, W2, b2, g2, be2):
    raise NotImplementedError("write your pallas kernel here")


# speedup vs baseline: 8.1628x; 8.1628x over previous
import functools
import jax
import jax.numpy as jnp
from jax.experimental import pallas as pl

_B = 8
_N = 4096
_S = 1024
_R = 0.2
_NS = 64
_CIN = 67
_CPAD = 80
_P = _B * _S * _NS
_EPS = 1e-5


def _fps_body(xyzt_ref, out_ref):
    X0 = xyzt_ref[:, 0, :]
    X1 = xyzt_ref[:, 1, :]
    X2 = xyzt_ref[:, 2, :]
    iota_n = jax.lax.broadcasted_iota(jnp.int32, (_B, _N), 1)

    def body(i, state):
        farthest, distance = state
        oh = (iota_n == farthest).astype(jnp.float32)
        c0 = jnp.sum(X0 * oh, axis=1, keepdims=True)
        c1 = jnp.sum(X1 * oh, axis=1, keepdims=True)
        c2 = jnp.sum(X2 * oh, axis=1, keepdims=True)
        out_ref[pl.ds(i, 1), :, :] = jnp.concatenate([c0, c1, c2], axis=1).reshape(1, _B, 3)
        d = (X0 - c0) ** 2 + (X1 - c1) ** 2 + (X2 - c2) ** 2
        distance = jnp.minimum(distance, d)
        m = jnp.max(distance, axis=1, keepdims=True)
        cand = jnp.where(distance == m, iota_n, _N)
        farthest = jnp.min(cand, axis=1, keepdims=True)
        return farthest, distance

    far0 = jnp.zeros((_B, 1), dtype=jnp.int32)
    dist0 = jnp.full((_B, _N), 1e10, dtype=jnp.float32)
    jax.lax.fori_loop(0, _S, body, (far0, dist0))


def _fps(xyzt):
    return pl.pallas_call(
        _fps_body,
        out_shape=jax.ShapeDtypeStruct((_S, _B, 3), jnp.float32),
    )(xyzt)


_SBLK = 8


def _cumsum_lanes(x):
    c = x
    k = 1
    while k < _N:
        z = jnp.zeros(c.shape[:-1] + (k,), dtype=c.dtype)
        c = c + jnp.concatenate([z, c[..., : _N - k]], axis=-1)
        k *= 2
    return c


def _l0_body(xyzt_ref, nxyz_ref, tbl_ref, w_ref, b_ref, h_ref, st_ref):
    b_id = pl.program_id(0)
    s_id = pl.program_id(1)

    X0 = xyzt_ref[0, 0:1, :]
    X1 = xyzt_ref[0, 1:2, :]
    X2 = xyzt_ref[0, 2:3, :]
    nx = nxyz_ref[0]
    n0 = nx[:, 0:1]
    n1 = nx[:, 1:2]
    n2 = nx[:, 2:3]
    D = (n0 - X0) ** 2 + (n1 - X1) ** 2 + (n2 - X2) ** 2
    mask = jnp.logical_not(D > _R * _R)
    rank = _cumsum_lanes(mask.astype(jnp.int32))
    count = rank[:, _N - 1:_N]

    jots = jax.lax.broadcasted_iota(jnp.int32, (_NS, _N), 0)
    jcol = jax.lax.broadcasted_iota(jnp.int32, (_NS, 1), 0)
    lane80 = jax.lax.broadcasted_iota(jnp.int32, (1, _CPAD), 1)
    scl = jnp.where(lane80 < 3, 1.0 / _R, 1.0)
    tbl = tbl_ref[0]

    hs = []
    for r in range(_SBLK):
        sel = (rank[r:r + 1, :] == jots + 1) & mask[r:r + 1, :]
        C = jnp.where(sel, 1.0, 0.0)
        pad = (jcol >= count[r, 0]).astype(jnp.float32)
        Cfix = C + pad * C[0:1, :]
        grouped = jnp.dot(Cfix, tbl, preferred_element_type=jnp.float32)
        offs = jnp.concatenate(
            [nx[r:r + 1, :], jnp.zeros((1, _CPAD - 3), jnp.float32)], axis=1)
        x = (grouped - offs) * scl
        h = jnp.dot(x, w_ref[:], preferred_element_type=jnp.float32) + b_ref[:]
        h_ref[pl.ds(r * _NS, _NS), :] = h
        hs.append(h)

    hcat = jnp.concatenate(hs, axis=0)
    ssum = jnp.sum(hcat, axis=0, keepdims=True)
    ssq = jnp.sum(hcat * hcat, axis=0, keepdims=True)
    upd = jnp.concatenate([ssum, ssq, jnp.zeros((6, ssum.shape[1]), jnp.float32)], axis=0)

    @pl.when(jnp.logical_and(b_id == 0, s_id == 0))
    def _():
        st_ref[:] = jnp.zeros_like(st_ref)

    st_ref[:] = st_ref[:] + upd


def _layer0(xyzt, new_xyz, tbl, w0tp, b0r):
    grid = (_B, _S // _SBLK)
    cout = w0tp.shape[1]
    return pl.pallas_call(
        _l0_body,
        grid=grid,
        in_specs=[
            pl.BlockSpec((1, 3, _N), lambda b, s: (b, 0, 0)),
            pl.BlockSpec((1, _SBLK, 3), lambda b, s: (b, s, 0)),
            pl.BlockSpec((1, _N, _CPAD), lambda b, s: (b, 0, 0)),
            pl.BlockSpec((_CPAD, cout), lambda b, s: (0, 0)),
            pl.BlockSpec((1, cout), lambda b, s: (0, 0)),
        ],
        out_specs=[
            pl.BlockSpec((_SBLK * _NS, cout), lambda b, s: (b * (_S // _SBLK) + s, 0)),
            pl.BlockSpec((8, cout), lambda b, s: (0, 0)),
        ],
        out_shape=[
            jax.ShapeDtypeStruct((_P, cout), jnp.float32),
            jax.ShapeDtypeStruct((8, cout), jnp.float32),
        ],
    )(xyzt, new_xyz, tbl, w0tp, b0r)


_TILE = 2048


def _mid_body(h_ref, st_ref, g_ref, be_ref, w_ref, b_ref, o_ref, sto_ref):
    mean = st_ref[0:1, :] / _P
    var = st_ref[1:2, :] / _P - mean * mean
    scale = g_ref[:] * jax.lax.rsqrt(var + _EPS)
    shift = be_ref[:] - mean * scale
    x = jnp.maximum(h_ref[:] * scale + shift, 0.0)
    h = jnp.dot(x, w_ref[:], preferred_element_type=jnp.float32) + b_ref[:]
    o_ref[:] = h
    ssum = jnp.sum(h, axis=0, keepdims=True)
    ssq = jnp.sum(h * h, axis=0, keepdims=True)
    upd = jnp.concatenate([ssum, ssq, jnp.zeros((6, ssum.shape[1]), jnp.float32)], axis=0)

    @pl.when(pl.program_id(0) == 0)
    def _():
        sto_ref[:] = jnp.zeros_like(sto_ref)

    sto_ref[:] = sto_ref[:] + upd


def _mid_layer(h, st, g, be, wt, br, cout):
    cin = h.shape[1]
    grid = (_P // _TILE,)
    return pl.pallas_call(
        _mid_body,
        grid=grid,
        in_specs=[
            pl.BlockSpec((_TILE, cin), lambda i: (i, 0)),
            pl.BlockSpec((8, cin), lambda i: (0, 0)),
            pl.BlockSpec((1, cin), lambda i: (0, 0)),
            pl.BlockSpec((1, cin), lambda i: (0, 0)),
            pl.BlockSpec((cin, cout), lambda i: (0, 0)),
            pl.BlockSpec((1, cout), lambda i: (0, 0)),
        ],
        out_specs=[
            pl.BlockSpec((_TILE, cout), lambda i: (i, 0)),
            pl.BlockSpec((8, cout), lambda i: (0, 0)),
        ],
        out_shape=[
            jax.ShapeDtypeStruct((_P, cout), jnp.float32),
            jax.ShapeDtypeStruct((8, cout), jnp.float32),
        ],
    )(h, st, g, be, wt, br)


def _final_body(h_ref, st_ref, g_ref, be_ref, o_ref):
    mean = st_ref[0:1, :] / _P
    var = st_ref[1:2, :] / _P - mean * mean
    scale = g_ref[:] * jax.lax.rsqrt(var + _EPS)
    shift = be_ref[:] - mean * scale
    x = jnp.maximum(h_ref[:] * scale + shift, 0.0)
    xg = x.reshape(_TILE // _NS, _NS, x.shape[1])
    o_ref[:] = jnp.max(xg, axis=1)


def _final_layer(h, st, g, be):
    cin = h.shape[1]
    grid = (_P // _TILE,)
    return pl.pallas_call(
        _final_body,
        grid=grid,
        in_specs=[
            pl.BlockSpec((_TILE, cin), lambda i: (i, 0)),
            pl.BlockSpec((8, cin), lambda i: (0, 0)),
            pl.BlockSpec((1, cin), lambda i: (0, 0)),
            pl.BlockSpec((1, cin), lambda i: (0, 0)),
        ],
        out_specs=pl.BlockSpec((_TILE // _NS, cin), lambda i: (i, 0)),
        out_shape=jax.ShapeDtypeStruct((_P // _NS, cin), jnp.float32),
    )(h, st, g, be)


@jax.jit
def kernel(xyz, points, W0, b0, g0, be0, W1, b1, g1, be1, W2, b2, g2, be2):
    xyzt = jnp.transpose(xyz, (0, 2, 1))
    nxt = _fps(xyzt)
    new_xyz = jnp.transpose(nxt, (1, 0, 2))

    tbl = jnp.concatenate(
        [xyz, points, jnp.zeros((_B, _N, _CPAD - _CIN), jnp.float32)], axis=-1)
    w0tp = jnp.concatenate(
        [W0.T, jnp.zeros((_CPAD - _CIN, W0.shape[0]), jnp.float32)], axis=0)

    h0, st0 = _layer0(xyzt, new_xyz, tbl, w0tp, b0.reshape(1, -1))
    h1, st1 = _mid_layer(h0, st0, g0.reshape(1, -1), be0.reshape(1, -1),
                         W1.T, b1.reshape(1, -1), W1.shape[0])
    h2, st2 = _mid_layer(h1, st1, g1.reshape(1, -1), be1.reshape(1, -1),
                         W2.T, b2.reshape(1, -1), W2.shape[0])
    np_flat = _final_layer(h2, st2, g2.reshape(1, -1), be2.reshape(1, -1))
    new_points = np_flat.reshape(_B, _S, W2.shape[0])
    return (new_xyz, new_points)
